# baseline (speedup 1.0000x reference)
import jax
import jax.numpy as jnp
from jax.experimental import pallas as pl
from jax.experimental.pallas import tpu as pltpu

_N_NODES = 10000
_N_GRAPHS = 64
_EPS = 1e-5
_BLK = 5000


def _fused_kernel(x_ref, batch_ref, Wa_ref, ba_ref, g_ref, be_ref, Wg_ref,
                  bg_ref, out_ref, acc_ref, csq_ref, cnt_ref):
    i = pl.program_id(0)
    nblk = pl.num_programs(0)

    x = x_ref[...]
    y = jax.lax.dot_general(x, Wa_ref[...], (((1,), (0,)), ((), ())),
                            preferred_element_type=jnp.float32)
    y = jnp.maximum(y + ba_ref[...], 0.0)

    b = batch_ref[0]
    seg_ids = jax.lax.broadcasted_iota(jnp.int32, (_N_GRAPHS + 1, 1), 0)
    A = ((b == seg_ids) | (seg_ids == _N_GRAPHS)).astype(jnp.float32)
    M1 = jax.lax.dot_general(A, y, (((1,), (0,)), ((), ())),
                             preferred_element_type=jnp.float32)
    csq = jnp.sum(y * y, axis=0, keepdims=True)
    cnt = jnp.sum(A[:_N_GRAPHS], axis=1, keepdims=True)

    @pl.when(i == 0)
    def _init():
        acc_ref[...] = M1
        csq_ref[...] = csq
        cnt_ref[...] = cnt

    @pl.when(i > 0)
    def _acc():
        acc_ref[...] += M1
        csq_ref[...] += csq
        cnt_ref[...] += cnt

    @pl.when(i == nblk - 1)
    def _finish():
        M = acc_ref[...]
        segsum = M[:_N_GRAPHS]
        colsum = M[_N_GRAPHS:]
        counts = cnt_ref[...]
        mean = colsum / _N_NODES
        var = csq_ref[...] / _N_NODES - mean * mean
        scale = g_ref[...] / jnp.sqrt(var + _EPS)
        seg_atom = (segsum - counts * mean) * scale + counts * be_ref[...]
        num = jax.lax.dot_general(seg_atom, Wg_ref[...],
                                  (((1,), (0,)), ((), ())),
                                  preferred_element_type=jnp.float32)
        num = num + counts * bg_ref[...]
        out_ref[...] = num / jnp.maximum(counts, 1.0)


def kernel(x, pair_features, W_atom, b_atom, g_atom, be_atom, W_pair, b_pair,
           g_pair, be_pair, W_a2p, b_a2p, W_g, b_g, pair_index, batch):
    del pair_features, W_pair, b_pair, g_pair, be_pair, W_a2p, b_a2p, pair_index
    d = x.shape[1]
    nblk = _N_NODES // _BLK
    batch3d = batch.astype(jnp.int32).reshape(nblk, 1, _BLK)
    out = pl.pallas_call(
        _fused_kernel,
        grid=(nblk,),
        in_specs=[
            pl.BlockSpec((_BLK, d), lambda i: (i, 0)),
            pl.BlockSpec((1, 1, _BLK), lambda i: (i, 0, 0)),
            pl.BlockSpec((d, d), lambda i: (0, 0)),
            pl.BlockSpec((1, d), lambda i: (0, 0)),
            pl.BlockSpec((1, d), lambda i: (0, 0)),
            pl.BlockSpec((1, d), lambda i: (0, 0)),
            pl.BlockSpec((d, d), lambda i: (0, 0)),
            pl.BlockSpec((1, d), lambda i: (0, 0)),
        ],
        out_specs=pl.BlockSpec((_N_GRAPHS, d), lambda i: (0, 0)),
        out_shape=jax.ShapeDtypeStruct((_N_GRAPHS, d), jnp.float32),
        scratch_shapes=[
            pltpu.VMEM((_N_GRAPHS + 1, d), jnp.float32),
            pltpu.VMEM((1, d), jnp.float32),
            pltpu.VMEM((_N_GRAPHS, 1), jnp.float32),
        ],
    )(x, batch3d, W_atom, b_atom.reshape(1, -1), g_atom.reshape(1, -1),
      be_atom.reshape(1, -1), W_g, b_g.reshape(1, -1))
    return out


# speedup vs baseline: 1.0068x; 1.0068x over previous
import jax
import jax.numpy as jnp
from jax.experimental import pallas as pl
from jax.experimental.pallas import tpu as pltpu

_N_NODES = 10000
_N_GRAPHS = 64
_EPS = 1e-5
_H = _N_NODES // 2


def _fused_kernel(x0_ref, x1_ref, batch_ref, Wa_ref, ba_ref, g_ref, be_ref,
                  Wg_ref, bg_ref, out_ref):
    Wa = Wa_ref[...]
    ba = ba_ref[...]
    seg_ids = jax.lax.broadcasted_iota(jnp.int32, (_N_GRAPHS + 1, 1), 0)

    M1 = jnp.zeros((_N_GRAPHS + 1, 128), jnp.float32)
    csq = jnp.zeros((1, 128), jnp.float32)
    counts = jnp.zeros((_N_GRAPHS, 1), jnp.float32)
    for half, x_ref in enumerate((x0_ref, x1_ref)):
        x = x_ref[...]
        y = jax.lax.dot_general(x, Wa, (((1,), (0,)), ((), ())),
                                preferred_element_type=jnp.float32)
        y = jnp.maximum(y + ba, 0.0)
        b = batch_ref[half:half + 1, :]
        A = ((b == seg_ids) | (seg_ids == _N_GRAPHS)).astype(jnp.float32)
        M1 = M1 + jax.lax.dot_general(A, y, (((1,), (0,)), ((), ())),
                                      preferred_element_type=jnp.float32)
        csq = csq + jnp.sum(y * y, axis=0, keepdims=True)
        counts = counts + jnp.sum(A[:_N_GRAPHS], axis=1, keepdims=True)

    segsum = M1[:_N_GRAPHS]
    colsum = M1[_N_GRAPHS:]
    mean = colsum / _N_NODES
    var = csq / _N_NODES - mean * mean
    scale = g_ref[...] / jnp.sqrt(var + _EPS)
    seg_atom = (segsum - counts * mean) * scale + counts * be_ref[...]
    num = jax.lax.dot_general(seg_atom, Wg_ref[...], (((1,), (0,)), ((), ())),
                              preferred_element_type=jnp.float32)
    num = num + counts * bg_ref[...]
    out_ref[...] = num / jnp.maximum(counts, 1.0)


def kernel(x, pair_features, W_atom, b_atom, g_atom, be_atom, W_pair, b_pair,
           g_pair, be_pair, W_a2p, b_a2p, W_g, b_g, pair_index, batch):
    del pair_features, W_pair, b_pair, g_pair, be_pair, W_a2p, b_a2p, pair_index
    d = x.shape[1]
    batch2d = batch.astype(jnp.int32).reshape(2, _H)
    out = pl.pallas_call(
        _fused_kernel,
        grid=(1,),
        in_specs=[
            pl.BlockSpec((_H, d), lambda i: (0, 0)),
            pl.BlockSpec((_H, d), lambda i: (1, 0)),
            pl.BlockSpec((2, _H), lambda i: (0, 0)),
            pl.BlockSpec((d, d), lambda i: (0, 0)),
            pl.BlockSpec((1, d), lambda i: (0, 0)),
            pl.BlockSpec((1, d), lambda i: (0, 0)),
            pl.BlockSpec((1, d), lambda i: (0, 0)),
            pl.BlockSpec((d, d), lambda i: (0, 0)),
            pl.BlockSpec((1, d), lambda i: (0, 0)),
        ],
        out_specs=pl.BlockSpec((_N_GRAPHS, d), lambda i: (0, 0)),
        out_shape=jax.ShapeDtypeStruct((_N_GRAPHS, d), jnp.float32),
    )(x, x, batch2d, W_atom, b_atom.reshape(1, -1), g_atom.reshape(1, -1),
      be_atom.reshape(1, -1), W_g, b_g.reshape(1, -1))
    return out


# speedup vs baseline: 1.0576x; 1.0504x over previous
import jax
import jax.numpy as jnp
from jax.experimental import pallas as pl
from jax.experimental.pallas import tpu as pltpu

_N_NODES = 10000
_N_GRAPHS = 64
_EPS = 1e-5


def _fused_kernel(x_ref, batch_ref, Wa_ref, ba_ref, g_ref, be_ref, Wg_ref,
                  bg_ref, out_ref):
    x = x_ref[...]
    y = jax.lax.dot_general(x, Wa_ref[...], (((1,), (0,)), ((), ())),
                            preferred_element_type=jnp.float32)
    y = jnp.maximum(y + ba_ref[...], 0.0)

    b = batch_ref[...]
    seg_ids = jax.lax.broadcasted_iota(jnp.int32, (_N_GRAPHS + 1, 1), 0)
    A = ((b == seg_ids) | (seg_ids == _N_GRAPHS)).astype(jnp.float32)
    M1 = jax.lax.dot_general(A, y, (((1,), (0,)), ((), ())),
                             preferred_element_type=jnp.float32)
    segsum = M1[:_N_GRAPHS]
    colsum = M1[_N_GRAPHS:]
    colsumsq = jnp.sum(y * y, axis=0, keepdims=True)
    counts = jnp.sum(A[:_N_GRAPHS], axis=1, keepdims=True)

    mean = colsum / _N_NODES
    var = colsumsq / _N_NODES - mean * mean
    scale = g_ref[...] / jnp.sqrt(var + _EPS)

    seg_atom = (segsum - counts * mean) * scale + counts * be_ref[...]
    num = jax.lax.dot_general(seg_atom, Wg_ref[...], (((1,), (0,)), ((), ())),
                              preferred_element_type=jnp.float32)
    num = num + counts * bg_ref[...]
    out_ref[...] = num / jnp.maximum(counts, 1.0)


def kernel(x, pair_features, W_atom, b_atom, g_atom, be_atom, W_pair, b_pair,
           g_pair, be_pair, W_a2p, b_a2p, W_g, b_g, pair_index, batch):
    del pair_features, W_pair, b_pair, g_pair, be_pair, W_a2p, b_a2p, pair_index
    batch2d = batch.astype(jnp.int32).reshape(1, _N_NODES)
    out = pl.pallas_call(
        _fused_kernel,
        out_shape=jax.ShapeDtypeStruct((_N_GRAPHS, x.shape[1]), jnp.float32),
    )(x, batch2d, W_atom, b_atom.reshape(1, -1), g_atom.reshape(1, -1),
      be_atom.reshape(1, -1), W_g, b_g.reshape(1, -1))
    return out
